# baseline (speedup 1.0000x reference)
import jax
import jax.numpy as jnp
import numpy as np
from jax.experimental import pallas as pl
from jax.experimental.pallas import tpu as pltpu

_N = 20000
_C = 91
_IMG_H = 800.0
_IMG_W = 1066.0
_SCORE_THRESH = 0.05
_NMS_THRESH = 0.5
_DET = 100
_KPRE = 2000
_CLIP = float(np.log(1000.0 / 16.0))

_BR = 400


def _dense_body(lg_ref, dx_ref, dy_ref, dw_ref, dh_ref, pr_ref,
                ms_ref, bx1_ref, by1_ref, bx2_ref, by2_ref):
    logits = lg_ref[...]
    m = jnp.max(logits, axis=-1, keepdims=True)
    e = jnp.exp(logits - m)
    scores = (e / jnp.sum(e, axis=-1, keepdims=True))[:, 1:]

    px1 = pr_ref[:, 0]
    py1 = pr_ref[:, 1]
    px2 = pr_ref[:, 2]
    py2 = pr_ref[:, 3]
    widths = px2 - px1
    heights = py2 - py1
    ctr_x = px1 + 0.5 * widths
    ctr_y = py1 + 0.5 * heights

    dx = dx_ref[...] * (1.0 / 10.0)
    dy = dy_ref[...] * (1.0 / 10.0)
    dw = jnp.minimum(dw_ref[...] * (1.0 / 5.0), _CLIP)
    dh = jnp.minimum(dh_ref[...] * (1.0 / 5.0), _CLIP)

    pcx = dx * widths[:, None] + ctr_x[:, None]
    pcy = dy * heights[:, None] + ctr_y[:, None]
    pw = jnp.exp(dw) * widths[:, None]
    ph = jnp.exp(dh) * heights[:, None]

    bx1 = jnp.clip(pcx - 0.5 * pw, 0.0, _IMG_W)
    by1 = jnp.clip(pcy - 0.5 * ph, 0.0, _IMG_H)
    bx2 = jnp.clip(pcx + 0.5 * pw, 0.0, _IMG_W)
    by2 = jnp.clip(pcy + 0.5 * ph, 0.0, _IMG_H)

    ws = bx2 - bx1
    hs = by2 - by1
    valid = (scores > _SCORE_THRESH) & (ws >= 1e-2) & (hs >= 1e-2)
    ms_ref[...] = jnp.where(valid, scores, -1.0)
    bx1_ref[...] = bx1
    by1_ref[...] = by1
    bx2_ref[...] = bx2
    by2_ref[...] = by2


def _dense_stage(class_logits, box_regression, proposals):
    reg = box_regression
    dx = reg[:, 4::4]
    dy = reg[:, 5::4]
    dw = reg[:, 6::4]
    dh = reg[:, 7::4]
    lg = class_logits

    grid = (_N // _BR,)
    row_spec = pl.BlockSpec((_BR, 90), lambda i: (i, 0))
    out_shape = jax.ShapeDtypeStruct((_N, 90), jnp.float32)
    outs = pl.pallas_call(
        _dense_body,
        grid=grid,
        in_specs=[
            pl.BlockSpec((_BR, _C), lambda i: (i, 0)),
            row_spec, row_spec, row_spec, row_spec,
            pl.BlockSpec((_BR, 4), lambda i: (i, 0)),
        ],
        out_specs=[row_spec] * 5,
        out_shape=[out_shape] * 5,
    )(lg, dx, dy, dw, dh, proposals)
    return outs


_KP = 2048
_NB = 128
_NBLK = _KP // _NB


def _nms_body(x1c, y1c, x2c, y2c, x1r, y1r, x2r, y2r, keep_ref,
              iou_s, bb_s, kb_s):
    ax1 = x1r[...]
    ay1 = y1r[...]
    ax2 = x2r[...]
    ay2 = y2r[...]
    area_r = jnp.maximum(ax2 - ax1, 0.0) * jnp.maximum(ay2 - ay1, 0.0)
    keep_ref[...] = jnp.ones((1, _KP), jnp.float32)
    lane = jax.lax.broadcasted_iota(jnp.int32, (1, _NB), 1)
    col = jax.lax.broadcasted_iota(jnp.int32, (1, _KP), 1)

    for bi in range(_NBLK):
        s = bi * _NB
        xb1 = x1c[s:s + _NB, :]
        yb1 = y1c[s:s + _NB, :]
        xb2 = x2c[s:s + _NB, :]
        yb2 = y2c[s:s + _NB, :]
        area_b = jnp.maximum(xb2 - xb1, 0.0) * jnp.maximum(yb2 - yb1, 0.0)
        xx1 = jnp.maximum(xb1, ax1)
        yy1 = jnp.maximum(yb1, ay1)
        xx2 = jnp.minimum(xb2, ax2)
        yy2 = jnp.minimum(yb2, ay2)
        inter = jnp.maximum(xx2 - xx1, 0.0) * jnp.maximum(yy2 - yy1, 0.0)
        iou = inter / (area_b + area_r - inter + 1e-9)
        hit = (iou > _NMS_THRESH).astype(jnp.float32)
        iou_s[...] = hit
        bb_s[...] = hit[:, s:s + _NB]
        kb_s[...] = keep_ref[0:1, s:s + _NB]

        def intra(i, _):
            row = bb_s[pl.ds(i, 1), :]
            kb = kb_s[...]
            ki = jnp.max(jnp.where(lane == i, kb, 0.0))
            sup = (row > 0.0) & (lane > i) & (ki > 0.0)
            kb_s[...] = jnp.where(sup, 0.0, kb)
            return 0

        jax.lax.fori_loop(0, _NB, intra, 0, unroll=False)

        kept = kb_s[...]
        keep_ref[0:1, s:s + _NB] = kept
        sup_all = jax.lax.dot_general(
            kept, iou_s[...],
            dimension_numbers=(((1,), (0,)), ((), ())),
            preferred_element_type=jnp.float32)
        later = col >= (s + _NB)
        keep_ref[...] = jnp.where((sup_all > 0.0) & later, 0.0,
                                  keep_ref[...])


def _nms_keep_pallas(boxes):
    pad = _KP - _KPRE
    b = jnp.pad(boxes, ((0, pad), (0, 0)))
    cols = [b[:, i:i + 1] for i in range(4)]
    rows = [b[:, i].reshape(1, _KP) for i in range(4)]
    keep = pl.pallas_call(
        _nms_body,
        out_shape=jax.ShapeDtypeStruct((1, _KP), jnp.float32),
        scratch_shapes=[
            pltpu.VMEM((_NB, _KP), jnp.float32),
            pltpu.VMEM((_NB, _NB), jnp.float32),
            pltpu.VMEM((1, _NB), jnp.float32),
        ],
    )(*cols, *rows)
    return keep[0, :_KPRE] > 0.0


@jax.jit
def kernel(class_logits, box_regression, proposals):
    masked, bx1, by1, bx2, by2 = _dense_stage(
        class_logits, box_regression, proposals)
    masked_f = masked.reshape(-1)
    top_vals, top_idx = jax.lax.top_k(masked_f, _KPRE)
    sel_x1 = bx1.reshape(-1)[top_idx]
    sel_y1 = by1.reshape(-1)[top_idx]
    sel_x2 = bx2.reshape(-1)[top_idx]
    sel_y2 = by2.reshape(-1)[top_idx]
    sel_boxes = jnp.stack([sel_x1, sel_y1, sel_x2, sel_y2], axis=1)
    sel_labels = (top_idx % 90) + 1
    max_coord = jnp.max(sel_boxes)
    offsets = sel_labels.astype(jnp.float32) * (max_coord + 1.0)
    keep = _nms_keep_pallas(sel_boxes + offsets[:, None])
    final_scores = jnp.where(keep & (top_vals > _SCORE_THRESH), top_vals, -1.0)
    fvals, fidx = jax.lax.top_k(final_scores, _DET)
    out = jnp.concatenate([sel_boxes[fidx], fvals[:, None]], axis=1)
    return jnp.where((fvals > _SCORE_THRESH)[:, None], out, 0.0)
